# baseline (speedup 1.0000x reference)
import functools

import jax
import jax.numpy as jnp
from jax import lax
from jax.experimental import pallas as pl
from jax.experimental.pallas import tpu as pltpu
from jax.experimental.pallas import tpu_sc as plsc

VOCAB = 1000000
DIM = 64
SEQ = 200
BATCH = 1024

NC = 2
NS = 16
NW = NC * NS
NLANE = 16
DREG = DIM // NLANE
DPAD = 128

N_TCOL = 7813
FULL_TCOL = 7812
TP_STEPS = 245

SEQ_PER_W = BATCH // NW
CHUNK_SEQ = 8
CHUNK_ROWS = CHUNK_SEQ * SEQ
N_CHUNKS = SEQ_PER_W // CHUNK_SEQ


def _wid():
    return lax.axis_index("s") * NC + lax.axis_index("c")


TP_G = 3
TP_COLS = TP_G * 128
TP_ROWS = TP_COLS // 2
TP_NSTEP = FULL_TCOL // TP_G
TP_KMAX = 41


def _tp_pairs(in_v, out_v, n_pairs, row_idx):
    def pair_body(p, carry):
        for half in range(2):
            cols = jnp.full((NLANE,), 2 * p + half, dtype=jnp.int32)
            for cc in range(DREG):
                v = plsc.load_gather(in_v, [row_idx[cc], cols])
                out_v[p, pl.ds(half * DIM + cc * NLANE, NLANE)] = v
        return carry

    lax.fori_loop(0, n_pairs, pair_body, 0)


def _tp_body(tT_hbm, stg_hbm, in0, in1, out0, out1, tail_v,
             si0, si1, so0, so1):
    wid = _wid()
    ins = (in0, in1)
    outs = (out0, out1)
    sis = (si0, si1)
    sos = (so0, so1)
    lanes = jnp.arange(NLANE, dtype=jnp.int32)
    row_idx = tuple(lanes + NLANE * cc for cc in range(DREG))

    pltpu.async_copy(tT_hbm.at[:, pl.ds(wid * TP_COLS, TP_COLS)], in0, si0)

    def ring(k2, carry):
        for b in range(2):
            s = (2 * k2 + b) * NW + wid

            @pl.when(s < TP_NSTEP)
            def _step():
                pltpu.make_async_copy(
                    tT_hbm.at[:, pl.ds(0, TP_COLS)], ins[b], sis[b]
                ).wait()
                s_next = s + NW

                @pl.when(s_next < TP_NSTEP)
                def _prefetch():
                    pltpu.async_copy(
                        tT_hbm.at[:, pl.ds(s_next * TP_COLS, TP_COLS)],
                        ins[1 - b], sis[1 - b],
                    )

                @pl.when(k2 > 0)
                def _reclaim():
                    pltpu.make_async_copy(
                        outs[b], stg_hbm.at[pl.ds(0, TP_ROWS), :], sos[b]
                    ).wait()

                _tp_pairs(ins[b], outs[b], TP_ROWS, row_idx)
                pltpu.async_copy(
                    outs[b], stg_hbm.at[pl.ds(s * TP_ROWS, TP_ROWS), :], sos[b]
                )

        return carry

    lax.fori_loop(0, TP_KMAX, ring, 0)
    for b in range(2):
        pltpu.make_async_copy(
            outs[b], stg_hbm.at[pl.ds(0, TP_ROWS), :], sos[b]
        ).wait()

    @pl.when(wid == NW - 1)
    def _tail():
        pltpu.sync_copy(tT_hbm.at[:, pl.ds(FULL_TCOL * 128, DIM)], tail_v)
        _tp_pairs(tail_v, out0, 32, row_idx)
        pltpu.sync_copy(
            out0.at[pl.ds(0, 32), :],
            stg_hbm.at[pl.ds(TP_NSTEP * TP_ROWS, 32), :],
        )


def _gather_body(x_hbm, table_hbm, pos_hbm, out_hbm, idx_v, rows_v, pos_v, sem):
    wid = _wid()
    pltpu.sync_copy(pos_hbm, pos_v)

    def chunk_body(i, carry):
        base_row = (wid * SEQ_PER_W + i * CHUNK_SEQ) * SEQ
        pltpu.sync_copy(x_hbm.at[pl.ds(base_row, CHUNK_ROWS)], idx_v)
        pltpu.async_copy(table_hbm.at[idx_v], rows_v, sem).wait()

        def l_body(l, carry_l):
            def s_body(s, carry_s):
                r = s * SEQ + l
                for cc in range(DREG):
                    sl = pl.ds(cc * NLANE, NLANE)
                    rows_v[r, sl] = rows_v[r, sl] + pos_v[l, sl]
                return carry_s

            return lax.fori_loop(0, CHUNK_SEQ, s_body, carry_l)

        lax.fori_loop(0, SEQ, l_body, 0)
        pltpu.sync_copy(
            rows_v, out_hbm.at[pl.ds(base_row, CHUNK_ROWS), pl.ds(0, DIM)]
        )
        return carry

    lax.fori_loop(0, N_CHUNKS, chunk_body, 0)


@jax.jit
def kernel(x, table, pos_emb):
    x_flat = x.reshape(-1).astype(jnp.int32)
    mesh = plsc.VectorSubcoreMesh(core_axis_name="c", subcore_axis_name="s")

    transpose_k = functools.partial(
        pl.kernel,
        mesh=mesh,
        compiler_params=pltpu.CompilerParams(
            use_tc_tiling_on_sc=True, needs_layout_passes=False
        ),
        out_type=jax.ShapeDtypeStruct((VOCAB // 2, DPAD), jnp.float32),
        scratch_types=[
            pltpu.VMEM((DIM, TP_COLS), jnp.float32),
            pltpu.VMEM((DIM, TP_COLS), jnp.float32),
            pltpu.VMEM((TP_ROWS, DPAD), jnp.float32),
            pltpu.VMEM((TP_ROWS, DPAD), jnp.float32),
            pltpu.VMEM((DIM, DIM), jnp.float32),
            pltpu.SemaphoreType.DMA,
            pltpu.SemaphoreType.DMA,
            pltpu.SemaphoreType.DMA,
            pltpu.SemaphoreType.DMA,
        ],
    )(_tp_body)
    staging = transpose_k(table.T)

    gather_k = functools.partial(
        pl.kernel,
        mesh=mesh,
        compiler_params=pltpu.CompilerParams(use_tc_tiling_on_sc=False),
        out_type=jax.ShapeDtypeStruct((BATCH * SEQ, DPAD), jnp.float32),
        scratch_types=[
            pltpu.VMEM((CHUNK_ROWS,), jnp.int32),
            pltpu.VMEM((CHUNK_ROWS, DIM), jnp.float32),
            pltpu.VMEM((SEQ, DIM), jnp.float32),
            pltpu.SemaphoreType.DMA,
        ],
    )(_gather_body)
    out = gather_k(x_flat, staging.reshape(VOCAB, DIM), pos_emb)
    return out[:, :DIM].reshape(BATCH, SEQ, DIM)


# speedup vs baseline: 1.2092x; 1.2092x over previous
import functools

import jax
import jax.numpy as jnp
from jax import lax
from jax.experimental import pallas as pl
from jax.experimental.pallas import tpu as pltpu
from jax.experimental.pallas import tpu_sc as plsc

VOCAB = 1000000
DIM = 64
SEQ = 200
BATCH = 1024

NC = 2
NS = 16
NW = NC * NS
NLANE = 16
DREG = DIM // NLANE
DPAD = 128

N_TCOL = 7813
FULL_TCOL = 7812
TP_STEPS = 245

SEQ_PER_W = BATCH // NW
CHUNK_SEQ = 8
CHUNK_ROWS = CHUNK_SEQ * SEQ
N_CHUNKS = SEQ_PER_W // CHUNK_SEQ


def _wid():
    return lax.axis_index("s") * NC + lax.axis_index("c")


TP_G = 3
TP_COLS = TP_G * 128
TP_ROWS = TP_COLS // 2
TP_NSTEP = FULL_TCOL // TP_G
TP_KMAX = 41


def _tp_slab(in_v, out_v, n_i):
    ngroups = n_i // NLANE
    lanes = jnp.arange(NLANE, dtype=jnp.int32)
    rowpat = lanes // 2
    colpat = (lanes & 1) * DIM
    rows_g = [rowpat + 8 * g for g in range(ngroups)]

    def d_body(d, carry):
        col_idx = colpat + d
        for g in range(ngroups):
            v = in_v[d, pl.ds(NLANE * g, NLANE)]
            plsc.store_scatter(out_v, [rows_g[g], col_idx], v)
        return carry

    lax.fori_loop(0, DIM, d_body, 0)


def _tp_body(tT_hbm, stg_hbm, in0, in1, out0, out1, tail_v,
             si0, si1, so0, so1):
    wid = _wid()
    ins = (in0, in1)
    outs = (out0, out1)
    sis = (si0, si1)
    sos = (so0, so1)
    pltpu.async_copy(tT_hbm.at[:, pl.ds(wid * TP_COLS, TP_COLS)], in0, si0)

    def ring(k2, carry):
        for b in range(2):
            s = (2 * k2 + b) * NW + wid

            @pl.when(s < TP_NSTEP)
            def _step():
                pltpu.make_async_copy(
                    tT_hbm.at[:, pl.ds(0, TP_COLS)], ins[b], sis[b]
                ).wait()
                s_next = s + NW

                @pl.when(s_next < TP_NSTEP)
                def _prefetch():
                    pltpu.async_copy(
                        tT_hbm.at[:, pl.ds(s_next * TP_COLS, TP_COLS)],
                        ins[1 - b], sis[1 - b],
                    )

                @pl.when(k2 > 0)
                def _reclaim():
                    pltpu.make_async_copy(
                        outs[b], stg_hbm.at[pl.ds(0, TP_ROWS), :], sos[b]
                    ).wait()

                _tp_slab(ins[b], outs[b], TP_COLS)
                pltpu.async_copy(
                    outs[b], stg_hbm.at[pl.ds(s * TP_ROWS, TP_ROWS), :], sos[b]
                )

        return carry

    lax.fori_loop(0, TP_KMAX, ring, 0)
    for b in range(2):
        pltpu.make_async_copy(
            outs[b], stg_hbm.at[pl.ds(0, TP_ROWS), :], sos[b]
        ).wait()

    @pl.when(wid == NW - 1)
    def _tail():
        pltpu.sync_copy(tT_hbm.at[:, pl.ds(FULL_TCOL * 128, DIM)], tail_v)
        _tp_slab(tail_v, out0, DIM)
        pltpu.sync_copy(
            out0.at[pl.ds(0, 32), :],
            stg_hbm.at[pl.ds(TP_NSTEP * TP_ROWS, 32), :],
        )


def _gather_body(x_hbm, table_hbm, pos_hbm, out_hbm, idx_v, rows_v, pos_v, sem):
    wid = _wid()
    pltpu.sync_copy(pos_hbm, pos_v)

    def chunk_body(i, carry):
        base_row = (wid * SEQ_PER_W + i * CHUNK_SEQ) * SEQ
        pltpu.sync_copy(x_hbm.at[pl.ds(base_row, CHUNK_ROWS)], idx_v)
        pltpu.async_copy(table_hbm.at[idx_v], rows_v, sem).wait()

        def l_body(l, carry_l):
            def s_body(s, carry_s):
                r = s * SEQ + l
                for cc in range(DREG):
                    sl = pl.ds(cc * NLANE, NLANE)
                    rows_v[r, sl] = rows_v[r, sl] + pos_v[l, sl]
                return carry_s

            return lax.fori_loop(0, CHUNK_SEQ, s_body, carry_l)

        lax.fori_loop(0, SEQ, l_body, 0)
        pltpu.sync_copy(
            rows_v, out_hbm.at[pl.ds(base_row, CHUNK_ROWS), pl.ds(0, DIM)]
        )
        return carry

    lax.fori_loop(0, N_CHUNKS, chunk_body, 0)


@jax.jit
def kernel(x, table, pos_emb):
    x_flat = x.reshape(-1).astype(jnp.int32)
    mesh = plsc.VectorSubcoreMesh(core_axis_name="c", subcore_axis_name="s")

    transpose_k = functools.partial(
        pl.kernel,
        mesh=mesh,
        compiler_params=pltpu.CompilerParams(
            use_tc_tiling_on_sc=True, needs_layout_passes=False
        ),
        out_type=jax.ShapeDtypeStruct((VOCAB // 2, DPAD), jnp.float32),
        scratch_types=[
            pltpu.VMEM((DIM, TP_COLS), jnp.float32),
            pltpu.VMEM((DIM, TP_COLS), jnp.float32),
            pltpu.VMEM((TP_ROWS, DPAD), jnp.float32),
            pltpu.VMEM((TP_ROWS, DPAD), jnp.float32),
            pltpu.VMEM((DIM, DIM), jnp.float32),
            pltpu.SemaphoreType.DMA,
            pltpu.SemaphoreType.DMA,
            pltpu.SemaphoreType.DMA,
            pltpu.SemaphoreType.DMA,
        ],
    )(_tp_body)
    staging = transpose_k(table.T)

    gather_k = functools.partial(
        pl.kernel,
        mesh=mesh,
        compiler_params=pltpu.CompilerParams(use_tc_tiling_on_sc=False),
        out_type=jax.ShapeDtypeStruct((BATCH * SEQ, DPAD), jnp.float32),
        scratch_types=[
            pltpu.VMEM((CHUNK_ROWS,), jnp.int32),
            pltpu.VMEM((CHUNK_ROWS, DIM), jnp.float32),
            pltpu.VMEM((SEQ, DIM), jnp.float32),
            pltpu.SemaphoreType.DMA,
        ],
    )(_gather_body)
    out = gather_k(x_flat, staging.reshape(VOCAB, DIM), pos_emb)
    return out[:, :DIM].reshape(BATCH, SEQ, DIM)


# speedup vs baseline: 1.8159x; 1.5017x over previous
import functools

import jax
import jax.numpy as jnp
from jax import lax
from jax.experimental import pallas as pl
from jax.experimental.pallas import tpu as pltpu
from jax.experimental.pallas import tpu_sc as plsc

VOCAB = 1000000
DIM = 64
SEQ = 200
BATCH = 1024

NC = 2
NS = 16
NW = NC * NS
NLANE = 16
DREG = DIM // NLANE
DPAD = 128

N_TCOL = 7813
FULL_TCOL = 7812
TP_STEPS = 245

SEQ_PER_W = BATCH // NW
CHUNK_SEQ = 8
CHUNK_ROWS = CHUNK_SEQ * SEQ
N_CHUNKS = SEQ_PER_W // CHUNK_SEQ


def _wid():
    return lax.axis_index("s") * NC + lax.axis_index("c")


TP_G = 3
TP_COLS = TP_G * 128
TP_ROWS = TP_COLS // 2
TP_NSTEP = FULL_TCOL // TP_G
TP_KMAX = 41


PK_BLK = 4096


def _pack_tc_body(in_ref, out_ref):
    t = in_ref[...]
    t3 = t.reshape(PK_BLK // 2, 2, DIM)
    out_ref[:, 0:DIM] = t3[:, 0, :]
    out_ref[:, DIM:DPAD] = t3[:, 1, :]


def _tp_slab(in_v, out_v, n_i):
    ngroups = n_i // NLANE
    lanes = jnp.arange(NLANE, dtype=jnp.int32)
    rowpat = lanes // 2
    colpat = (lanes & 1) * DIM
    rows_g = [rowpat + 8 * g for g in range(ngroups)]

    def d_body(d, carry):
        col_idx = colpat + d
        for g in range(ngroups):
            v = in_v[d, pl.ds(NLANE * g, NLANE)]
            plsc.store_scatter(out_v, [rows_g[g], col_idx], v)
        return carry

    lax.fori_loop(0, DIM, d_body, 0)


def _tp_body(tT_hbm, stg_hbm, in0, in1, out0, out1, tail_v,
             si0, si1, so0, so1):
    wid = _wid()
    ins = (in0, in1)
    outs = (out0, out1)
    sis = (si0, si1)
    sos = (so0, so1)
    pltpu.async_copy(tT_hbm.at[:, pl.ds(wid * TP_COLS, TP_COLS)], in0, si0)

    def ring(k2, carry):
        for b in range(2):
            s = (2 * k2 + b) * NW + wid

            @pl.when(s < TP_NSTEP)
            def _step():
                pltpu.make_async_copy(
                    tT_hbm.at[:, pl.ds(0, TP_COLS)], ins[b], sis[b]
                ).wait()
                s_next = s + NW

                @pl.when(s_next < TP_NSTEP)
                def _prefetch():
                    pltpu.async_copy(
                        tT_hbm.at[:, pl.ds(s_next * TP_COLS, TP_COLS)],
                        ins[1 - b], sis[1 - b],
                    )

                @pl.when(k2 > 0)
                def _reclaim():
                    pltpu.make_async_copy(
                        outs[b], stg_hbm.at[pl.ds(0, TP_ROWS), :], sos[b]
                    ).wait()

                _tp_slab(ins[b], outs[b], TP_COLS)
                pltpu.async_copy(
                    outs[b], stg_hbm.at[pl.ds(s * TP_ROWS, TP_ROWS), :], sos[b]
                )

        return carry

    lax.fori_loop(0, TP_KMAX, ring, 0)
    for b in range(2):
        pltpu.make_async_copy(
            outs[b], stg_hbm.at[pl.ds(0, TP_ROWS), :], sos[b]
        ).wait()

    @pl.when(wid == NW - 1)
    def _tail():
        pltpu.sync_copy(tT_hbm.at[:, pl.ds(FULL_TCOL * 128, DIM)], tail_v)
        _tp_slab(tail_v, out0, DIM)
        pltpu.sync_copy(
            out0.at[pl.ds(0, 32), :],
            stg_hbm.at[pl.ds(TP_NSTEP * TP_ROWS, 32), :],
        )


def _gather_body(x_hbm, table_hbm, pos_hbm, out_hbm, idx_v, rows_v, pos_v, sem):
    wid = _wid()
    pltpu.sync_copy(pos_hbm, pos_v)

    def chunk_body(i, carry):
        base_row = (wid * SEQ_PER_W + i * CHUNK_SEQ) * SEQ
        pltpu.sync_copy(x_hbm.at[pl.ds(base_row, CHUNK_ROWS)], idx_v)
        pltpu.async_copy(table_hbm.at[idx_v], rows_v, sem).wait()

        def l_body(l, carry_l):
            def s_body(s, carry_s):
                r = s * SEQ + l
                for cc in range(DREG):
                    sl = pl.ds(cc * NLANE, NLANE)
                    rows_v[r, sl] = rows_v[r, sl] + pos_v[l, sl]
                return carry_s

            return lax.fori_loop(0, CHUNK_SEQ, s_body, carry_l)

        lax.fori_loop(0, SEQ, l_body, 0)
        pltpu.sync_copy(
            rows_v, out_hbm.at[pl.ds(base_row, CHUNK_ROWS), pl.ds(0, DIM)]
        )
        return carry

    lax.fori_loop(0, N_CHUNKS, chunk_body, 0)


@jax.jit
def kernel(x, table, pos_emb):
    x_flat = x.reshape(-1).astype(jnp.int32)
    mesh = plsc.VectorSubcoreMesh(core_axis_name="c", subcore_axis_name="s")

    n_blk = (VOCAB + PK_BLK - 1) // PK_BLK
    staging = pl.pallas_call(
        _pack_tc_body,
        grid=(n_blk,),
        in_specs=[pl.BlockSpec((PK_BLK, DIM), lambda i: (i, 0))],
        out_specs=pl.BlockSpec((PK_BLK // 2, DPAD), lambda i: (i, 0)),
        out_shape=jax.ShapeDtypeStruct((VOCAB // 2, DPAD), jnp.float32),
    )(table)

    gather_k = functools.partial(
        pl.kernel,
        mesh=mesh,
        compiler_params=pltpu.CompilerParams(use_tc_tiling_on_sc=False),
        out_type=jax.ShapeDtypeStruct((BATCH * SEQ, DPAD), jnp.float32),
        scratch_types=[
            pltpu.VMEM((CHUNK_ROWS,), jnp.int32),
            pltpu.VMEM((CHUNK_ROWS, DIM), jnp.float32),
            pltpu.VMEM((SEQ, DIM), jnp.float32),
            pltpu.SemaphoreType.DMA,
        ],
    )(_gather_body)
    out = gather_k(x_flat, staging.reshape(VOCAB, DIM), pos_emb)
    return out[:, :DIM].reshape(BATCH, SEQ, DIM)


# speedup vs baseline: 2.4228x; 1.3342x over previous
import functools

import jax
import jax.numpy as jnp
from jax import lax
from jax.experimental import pallas as pl
from jax.experimental.pallas import tpu as pltpu
from jax.experimental.pallas import tpu_sc as plsc

VOCAB = 1000000
DIM = 64
SEQ = 200
BATCH = 1024

NC = 2
NS = 16
NW = NC * NS
NLANE = 16
DREG = DIM // NLANE
DPAD = 128

SEQ_PER_W = BATCH // NW
CHUNK_SEQ = 8
CHUNK_ROWS = CHUNK_SEQ * SEQ
N_CHUNKS = SEQ_PER_W // CHUNK_SEQ


def _wid():
    return lax.axis_index("s") * NC + lax.axis_index("c")


TP_BLK = 2048


def _tp_tc_body(in_ref, out_ref):
    t = in_ref[...]
    tt = jnp.transpose(t, (1, 0))
    t3 = tt.reshape(TP_BLK // 2, 2, DIM)
    out_ref[:, 0:DIM] = t3[:, 0, :]
    out_ref[:, DIM:DPAD] = t3[:, 1, :]


def _gather_body(x_hbm, table_hbm, pos_hbm, out_hbm, idx_v, rows_v, pos_v, sem):
    wid = _wid()
    pltpu.sync_copy(pos_hbm, pos_v)

    def chunk_body(i, carry):
        base_row = (wid * SEQ_PER_W + i * CHUNK_SEQ) * SEQ
        pltpu.sync_copy(x_hbm.at[pl.ds(base_row, CHUNK_ROWS)], idx_v)
        pltpu.async_copy(table_hbm.at[idx_v], rows_v, sem).wait()

        def l_body(l, carry_l):
            def s_body(s, carry_s):
                r = s * SEQ + l
                for cc in range(DREG):
                    sl = pl.ds(cc * NLANE, NLANE)
                    rows_v[r, sl] = rows_v[r, sl] + pos_v[l, sl]
                return carry_s

            return lax.fori_loop(0, CHUNK_SEQ, s_body, carry_l)

        lax.fori_loop(0, SEQ, l_body, 0)
        pltpu.sync_copy(
            rows_v, out_hbm.at[pl.ds(base_row, CHUNK_ROWS), pl.ds(0, DIM)]
        )
        return carry

    lax.fori_loop(0, N_CHUNKS, chunk_body, 0)


@jax.jit
def kernel(x, table, pos_emb):
    x_flat = x.reshape(-1).astype(jnp.int32)
    mesh = plsc.VectorSubcoreMesh(core_axis_name="c", subcore_axis_name="s")

    n_blk = (VOCAB + TP_BLK - 1) // TP_BLK
    staging = pl.pallas_call(
        _tp_tc_body,
        grid=(n_blk,),
        in_specs=[pl.BlockSpec((DIM, TP_BLK), lambda i: (0, i))],
        out_specs=pl.BlockSpec((TP_BLK // 2, DPAD), lambda i: (i, 0)),
        out_shape=jax.ShapeDtypeStruct((VOCAB // 2, DPAD), jnp.float32),
    )(table.T)

    gather_k = functools.partial(
        pl.kernel,
        mesh=mesh,
        compiler_params=pltpu.CompilerParams(use_tc_tiling_on_sc=False),
        out_type=jax.ShapeDtypeStruct((BATCH * SEQ, DPAD), jnp.float32),
        scratch_types=[
            pltpu.VMEM((CHUNK_ROWS,), jnp.int32),
            pltpu.VMEM((CHUNK_ROWS, DIM), jnp.float32),
            pltpu.VMEM((SEQ, DIM), jnp.float32),
            pltpu.SemaphoreType.DMA,
        ],
    )(_gather_body)
    out = gather_k(x_flat, staging.reshape(VOCAB, DIM), pos_emb)
    return out[:, :DIM].reshape(BATCH, SEQ, DIM)
